# baseline (speedup 1.0000x reference)
import functools

import jax
import jax.numpy as jnp
from jax import lax
from jax.experimental import pallas as pl
from jax.experimental.pallas import tpu as pltpu
from jax.experimental.pallas import tpu_sc as plsc

_B, _A, _MB, _NC, _NV, _NE = 4096, 38, 64, 128, 10, 5
_R = _A * _NV + 2 * _MB * _A + _MB * _NE
_RPAD = _R + 4
_J = _A + 2 * _MB + _MB
_JPAD = 240
_JH = _JPAD // 2
_NW = 32
_NB = _B // _NW
_L = 16
_NCV = _NC // _L


def _prep_body(g_ref, vt_ref, ed_ref, et_ref, w_ref, card_ref,
               gt_ref, bias_ref):
    g = g_ref[:]
    gt = jnp.transpose(g)
    pad = jnp.zeros((_RPAD - _R, _NC), jnp.float32)
    gt_ref[:] = jnp.concatenate([gt, pad], axis=0)

    k_v = jax.nn.logsumexp(vt_ref[:], axis=-1).sum(-1)
    k_e = jax.nn.logsumexp(ed_ref[:], axis=-1).sum(-1)
    k_t = jax.nn.logsumexp(et_ref[:], axis=-1).sum(-1)
    w = w_ref[0, :]
    card = card_ref[:]
    card_s = card[_A - 1, _MB - 1] - jax.nn.logsumexp(
        jax.nn.logsumexp(card, axis=1))
    bias = w - jax.nn.logsumexp(w) - (k_v + k_e + k_t) + card_s
    bias_ref[0, :] = bias


_prep = pl.pallas_call(
    _prep_body,
    out_shape=[
        jax.ShapeDtypeStruct((_RPAD, _NC), jnp.float32),
        jax.ShapeDtypeStruct((1, _NC), jnp.float32),
    ],
)


def _finish_body(acc_ref, bias_ref, out_ref):
    x = acc_ref[:] + bias_ref[:]
    m = jnp.max(x, axis=1, keepdims=True)
    s = jnp.sum(jnp.exp(x - m), axis=1, keepdims=True)
    out_ref[:] = m + jnp.log(s)


_finish = pl.pallas_call(
    _finish_body,
    out_shape=jax.ShapeDtypeStruct((_B, 1), jnp.float32),
)


@functools.partial(
    pl.kernel,
    out_type=jax.ShapeDtypeStruct((_B, _NC), jnp.float32),
    mesh=plsc.VectorSubcoreMesh(core_axis_name="c", subcore_axis_name="s"),
    scratch_types=[
        pltpu.VMEM((_NB, 2, _JH), jnp.int32),
        pltpu.VMEM((_JPAD, _NC), jnp.float32),
        pltpu.VMEM((_NB, _NC), jnp.float32),
        pltpu.SemaphoreType.DMA,
    ],
)
def _sc_main(idx_hbm, gt_hbm, acc_hbm, idx_v, rows_v, acc_v, sem):
    wid = lax.axis_index("s") * 2 + lax.axis_index("c")
    base = wid * _NB
    pltpu.sync_copy(idx_hbm.at[pl.ds(base, _NB)], idx_v)

    def row_body(b, carry):
        cp0 = pltpu.async_copy(gt_hbm.at[idx_v.at[b, 0]],
                               rows_v.at[pl.ds(0, _JH)], sem)
        cp1 = pltpu.async_copy(gt_hbm.at[idx_v.at[b, 1]],
                               rows_v.at[pl.ds(_JH, _JH)], sem)
        cp0.wait()
        cp1.wait()

        def acc_body(j, acc):
            return tuple(acc[k] + rows_v[j, pl.ds(k * _L, _L)]
                         for k in range(_NCV))

        acc = lax.fori_loop(
            0, _JPAD, acc_body,
            tuple(jnp.zeros((_L,), jnp.float32) for _ in range(_NCV)))
        for k in range(_NCV):
            acc_v[b, pl.ds(k * _L, _L)] = acc[k]
        return carry

    lax.fori_loop(0, _NB, row_body, 0)
    pltpu.sync_copy(acc_v, acc_hbm.at[pl.ds(base, _NB)])


def kernel(v, e, logits_w, vtype_logits, edges_logits, etype_logits,
           card_logits):
    v = v.astype(jnp.int32)
    e = e.astype(jnp.int32)
    vtype = v[..., 1]
    edges = e[..., :2].reshape(_B, 2 * _MB)
    etype = e[..., 2]

    offs_v = (jnp.arange(_A, dtype=jnp.int32) * _NV)[None]
    offs_e = (_A * _NV + jnp.arange(2 * _MB, dtype=jnp.int32) * _A)[None]
    offs_t = (_A * _NV + 2 * _MB * _A
              + jnp.arange(_MB, dtype=jnp.int32) * _NE)[None]
    idx = jnp.concatenate(
        [vtype + offs_v, edges + offs_e, etype + offs_t], axis=1)
    idx = jnp.concatenate(
        [idx, jnp.full((_B, _JPAD - _J), _R, jnp.int32)], axis=1)
    idx = idx.reshape(_B, 2, _JH)

    g = jnp.concatenate([
        vtype_logits.reshape(_NC, _A * _NV),
        edges_logits.reshape(_NC, 2 * _MB * _A),
        etype_logits.reshape(_NC, _MB * _NE),
    ], axis=1)

    gt, bias2d = _prep(g, vtype_logits, edges_logits, etype_logits,
                       logits_w.reshape(1, _NC), card_logits)
    acc = _sc_main(idx, gt)
    out = _finish(acc, bias2d)
    return out.reshape(_B)


# speedup vs baseline: 7.4817x; 7.4817x over previous
import functools

import jax
import jax.numpy as jnp
from jax import lax
from jax.experimental import pallas as pl
from jax.experimental.pallas import tpu as pltpu
from jax.experimental.pallas import tpu_sc as plsc

_B, _A, _MB, _NC, _NV, _NE = 4096, 38, 64, 128, 10, 5
_R = _A * _NV + 2 * _MB * _A + _MB * _NE
_RPAD = _R + 4
_J = _A + 2 * _MB + _MB
_JPAD = 240
_JC = _JPAD // 2
_L = 16
_NP = _NC // 2
_CG = 4
_PP = _NP // _CG
_BG = 8
_NBR = _B // _BG
_NBG = _NBR // _L
_NG16 = _B // _L
_GW = _JPAD * _L


def _prep_body(g_ref, vt_ref, ed_ref, et_ref, w_ref, card_ref,
               t2_ref, bias_ref):
    g = g_ref[:]
    gp = jnp.concatenate([g, jnp.zeros((_NC, _RPAD - _R), jnp.float32)],
                         axis=1)
    lo = lax.bitcast_convert_type(
        gp[:_NP, :].astype(jnp.bfloat16), jnp.int16).astype(jnp.int32)
    hi = lax.bitcast_convert_type(
        gp[_NP:, :].astype(jnp.bfloat16), jnp.int16).astype(jnp.int32)
    t2_ref[:] = jnp.left_shift(hi, 16) | (lo & 0xFFFF)

    k_v = jax.nn.logsumexp(vt_ref[:], axis=-1).sum(-1)
    k_e = jax.nn.logsumexp(ed_ref[:], axis=-1).sum(-1)
    k_t = jax.nn.logsumexp(et_ref[:], axis=-1).sum(-1)
    w = w_ref[0, :]
    card = card_ref[:]
    card_s = card[_A - 1, _MB - 1] - jax.nn.logsumexp(
        jax.nn.logsumexp(card, axis=1))
    bias = w - jax.nn.logsumexp(w) - (k_v + k_e + k_t) + card_s
    bias_ref[0, :] = bias


_prep = pl.pallas_call(
    _prep_body,
    out_shape=[
        jax.ShapeDtypeStruct((_NP, _RPAD), jnp.int32),
        jax.ShapeDtypeStruct((1, _NC), jnp.float32),
    ],
)


def _finish_body(acc_ref, bias_ref, out_ref):
    x = acc_ref[:] + bias_ref[:]
    m = jnp.max(x, axis=0, keepdims=True)
    s = jnp.sum(jnp.exp(x - m), axis=0, keepdims=True)
    out_ref[:] = m + jnp.log(s)


_finish = pl.pallas_call(
    _finish_body,
    out_shape=jax.ShapeDtypeStruct((1, _B), jnp.float32),
)


@functools.partial(
    pl.kernel,
    out_type=jax.ShapeDtypeStruct((_NC, _B), jnp.float32),
    mesh=plsc.VectorSubcoreMesh(core_axis_name="c", subcore_axis_name="s"),
    compiler_params=pltpu.CompilerParams(needs_layout_passes=False),
    scratch_types=[
        pltpu.VMEM((_PP * _RPAD,), jnp.int32),
        pltpu.VMEM((_GW,), jnp.int32),
        pltpu.VMEM((2 * _PP, _NBR), jnp.float32),
        pltpu.SemaphoreType.DMA,
    ],
)
def _sc_main(idxg_hbm, t2_hbm, acct_hbm, tab_v, idx_v, out_v, sem):
    wid = lax.axis_index("s") * 2 + lax.axis_index("c")
    bg = wid // _CG
    cg = wid % _CG
    rbase = bg * _NBR
    gbase = bg * _NBG
    pltpu.sync_copy(t2_hbm.at[pl.ds(cg * _PP * _RPAD, _PP * _RPAD)], tab_v)

    def b16_body(t, carry):
        pltpu.sync_copy(idxg_hbm.at[pl.ds((gbase + t) * _GW, _GW)], idx_v)
        for jc in range(2):
            def j_body(j, accs):
                iv = idx_v[pl.ds((jc * _JC + j) * _L, _L)]
                return tuple(
                    accs[p] + plsc.bitcast(
                        plsc.load_gather(tab_v, [iv + (p * _RPAD)]),
                        jnp.bfloat16)
                    for p in range(_PP))

            accs = lax.fori_loop(
                0, _JC, j_body,
                tuple(jnp.zeros((2 * _L,), jnp.bfloat16)
                      for _ in range(_PP)))
            for p in range(_PP):
                flo, fhi = plsc.unpack(accs[p],
                                       format=plsc.PackFormat.INTERLEAVED)
                if jc == 0:
                    out_v[p, pl.ds(t * _L, _L)] = flo
                    out_v[p + _PP, pl.ds(t * _L, _L)] = fhi
                else:
                    out_v[p, pl.ds(t * _L, _L)] = (
                        out_v[p, pl.ds(t * _L, _L)] + flo)
                    out_v[p + _PP, pl.ds(t * _L, _L)] = (
                        out_v[p + _PP, pl.ds(t * _L, _L)] + fhi)
        return carry

    lax.fori_loop(0, _NBG, b16_body, 0)
    pltpu.sync_copy(out_v.at[pl.ds(0, _PP)],
                    acct_hbm.at[pl.ds(cg * _PP, _PP), pl.ds(rbase, _NBR)])
    pltpu.sync_copy(out_v.at[pl.ds(_PP, _PP)],
                    acct_hbm.at[pl.ds(_NP + cg * _PP, _PP),
                                pl.ds(rbase, _NBR)])


def kernel(v, e, logits_w, vtype_logits, edges_logits, etype_logits,
           card_logits):
    v = v.astype(jnp.int32)
    e = e.astype(jnp.int32)
    vtype = v[..., 1]
    edges = e[..., :2].reshape(_B, 2 * _MB)
    etype = e[..., 2]

    offs_v = (jnp.arange(_A, dtype=jnp.int32) * _NV)[None]
    offs_e = (_A * _NV + jnp.arange(2 * _MB, dtype=jnp.int32) * _A)[None]
    offs_t = (_A * _NV + 2 * _MB * _A
              + jnp.arange(_MB, dtype=jnp.int32) * _NE)[None]
    idx = jnp.concatenate(
        [vtype + offs_v, edges + offs_e, etype + offs_t], axis=1)
    idx = jnp.concatenate(
        [idx, jnp.full((_B, _JPAD - _J), _R, jnp.int32)], axis=1)
    idxg = jnp.transpose(idx.reshape(_NG16, _L, _JPAD),
                         (0, 2, 1)).reshape(-1)

    g = jnp.concatenate([
        vtype_logits.reshape(_NC, _A * _NV),
        edges_logits.reshape(_NC, 2 * _MB * _A),
        etype_logits.reshape(_NC, _MB * _NE),
    ], axis=1)

    t2, bias2d = _prep(g, vtype_logits, edges_logits, etype_logits,
                       logits_w.reshape(1, _NC), card_logits)
    acct = _sc_main(idxg, t2.reshape(-1))
    out = _finish(acct, bias2d.reshape(_NC, 1))
    return out.reshape(_B)
